# baseline (speedup 1.0000x reference)
import jax
import jax.numpy as jnp
from jax import lax
from jax.experimental import pallas as pl
from jax.experimental.pallas import tpu as pltpu
from jax.experimental.pallas import tpu_sc as plsc

N = 10000
E = 320000
D = 128

NC = 2
NS = 16
NW = NC * NS
CHUNK = 128
K = (-(-E // (NW * CHUNK)) + 7) // 8 * 8
EPT = K * CHUNK
E_PAD = NW * EPT
ROWS_PER_TILE = 640
NPAD = NS * ROWS_PER_TILE
F32 = jnp.float32
I32 = jnp.int32


def _segsum_call(with_deg):
    mesh = plsc.VectorSubcoreMesh(core_axis_name="c", subcore_axis_name="s")
    out_type = [jax.ShapeDtypeStruct((NC * NPAD, D), F32)]
    scratch = [
        pltpu.VMEM((2, CHUNK), I32),
        pltpu.VMEM((K, CHUNK), I32),
        pltpu.VMEM((2, CHUNK, D), F32),
        pltpu.VMEM_SHARED((NPAD, D), F32),
        pltpu.SemaphoreType.DMA((2,)),
    ]
    if with_deg:
        out_type.append(jax.ShapeDtypeStruct((NC * NPAD, D), F32))

    def body(rows_hbm, src_hbm, dst_hbm, zrows_hbm, orows_hbm, agg_out, *rest):
        if with_deg:
            deg_out, src_v, dst_all, rows_v, agg_sh, sem = rest
        else:
            src_v, dst_all, rows_v, agg_sh, sem = rest
        c = lax.axis_index("c")
        s = lax.axis_index("s")
        wid = s * NC + c

        pltpu.sync_copy(dst_hbm.at[pl.ds(wid * K, K)], dst_all)

        def fill(t, _):
            pltpu.sync_copy(rows_v.at[1],
                            agg_sh.at[pl.ds(s * ROWS_PER_TILE + t * CHUNK, CHUNK)])
            return 0

        def wb(out):
            def step(t, _):
                r0 = s * ROWS_PER_TILE + t * CHUNK
                pltpu.sync_copy(agg_sh.at[pl.ds(r0, CHUNK)],
                                out.at[pl.ds(c * NPAD + r0, CHUNK)])
                return 0
            lax.fori_loop(0, ROWS_PER_TILE // CHUNK, step, 0)

        pltpu.sync_copy(zrows_hbm, rows_v.at[1])
        lax.fori_loop(0, ROWS_PER_TILE // CHUNK, fill, 0)

        if with_deg:
            pltpu.sync_copy(orows_hbm, rows_v.at[0])
            plsc.subcore_barrier()

            def deg_chunk(j, _):
                pltpu.sync_copy(rows_v.at[0], agg_sh.at[dst_all.at[j]],
                                add=True)
                return 0
            lax.fori_loop(0, K, deg_chunk, 0)
            plsc.subcore_barrier()
            wb(deg_out)
            lax.fori_loop(0, ROWS_PER_TILE // CHUNK, fill, 0)

        plsc.subcore_barrier()

        pltpu.sync_copy(src_hbm.at[pl.ds(wid * K, 1)], src_v.at[pl.ds(0, 1)])
        pltpu.async_copy(rows_hbm.at[src_v.at[0]], rows_v.at[0], sem.at[0])

        def chunk_body(j, _):
            b = lax.rem(j, 2)
            nb = lax.rem(j + 1, 2)

            @pl.when(j + 1 < K)
            def _():
                pltpu.sync_copy(src_hbm.at[pl.ds(wid * K + j + 1, 1)],
                                src_v.at[pl.ds(nb, 1)])
                pltpu.async_copy(rows_hbm.at[src_v.at[nb]], rows_v.at[nb],
                                 sem.at[nb])
            pltpu.make_async_copy(rows_hbm.at[src_v.at[b]], rows_v.at[b],
                                  sem.at[b]).wait()
            pltpu.sync_copy(rows_v.at[b], agg_sh.at[dst_all.at[j]], add=True)
            return 0
        lax.fori_loop(0, K, chunk_body, 0)
        plsc.subcore_barrier()
        wb(agg_out)

    return pl.kernel(body, out_type=out_type, mesh=mesh, scratch_types=scratch)


def _dense_call(relu):
    BLK = 1024
    grid = NPAD // BLK
    half = NPAD // BLK

    def body(agg0_ref, agg1_ref, deg0_ref, deg1_ref, x_ref, wl_ref, wr_ref,
             b_ref, out_ref):
        agg = agg0_ref[...] + agg1_ref[...]
        deg = deg0_ref[:, :1] + deg1_ref[:, :1]
        mean = agg / jnp.maximum(deg, 1.0)
        h = (jnp.dot(mean, wl_ref[...], preferred_element_type=F32)
             + jnp.dot(x_ref[...], wr_ref[...], preferred_element_type=F32)
             + b_ref[...])
        out_ref[...] = jnp.maximum(h, 0.0) if relu else h

    return pl.pallas_call(
        body,
        grid=(grid,),
        in_specs=[
            pl.BlockSpec((BLK, D), lambda i: (i, 0)),
            pl.BlockSpec((BLK, D), lambda i: (half + i, 0)),
            pl.BlockSpec((BLK, D), lambda i: (i, 0)),
            pl.BlockSpec((BLK, D), lambda i: (half + i, 0)),
            pl.BlockSpec((BLK, D), lambda i: (i, 0)),
            pl.BlockSpec((D, D), lambda i: (0, 0)),
            pl.BlockSpec((D, D), lambda i: (0, 0)),
            pl.BlockSpec((1, D), lambda i: (0, 0)),
        ],
        out_specs=pl.BlockSpec((BLK, D), lambda i: (i, 0)),
        out_shape=jax.ShapeDtypeStruct((NPAD, D), F32),
    )


def _decode_call():
    mesh = plsc.VectorSubcoreMesh(core_axis_name="c", subcore_axis_name="s")
    scratch = [
        pltpu.VMEM((K, CHUNK), I32),
        pltpu.VMEM((K, CHUNK), I32),
        pltpu.VMEM((2, CHUNK, D), F32),
        pltpu.VMEM((2, CHUNK, D), F32),
        pltpu.VMEM((CHUNK, 16), F32),
        pltpu.SemaphoreType.DMA((2,)),
        pltpu.SemaphoreType.DMA((2,)),
    ]

    def body(z_hbm, src_hbm, dst_hbm, out_hbm, src_all, dst_all, srows, drows,
             acc_v, sems, semd):
        c = lax.axis_index("c")
        s = lax.axis_index("s")
        wid = s * NC + c

        pltpu.sync_copy(src_hbm.at[pl.ds(wid * K, K)], src_all)
        pltpu.sync_copy(dst_hbm.at[pl.ds(wid * K, K)], dst_all)

        pltpu.async_copy(z_hbm.at[src_all.at[0]], srows.at[0], sems.at[0])
        pltpu.async_copy(z_hbm.at[dst_all.at[0]], drows.at[0], semd.at[0])

        def chunk_body(j, _):
            b = lax.rem(j, 2)
            nb = lax.rem(j + 1, 2)

            @pl.when(j + 1 < K)
            def _():
                pltpu.async_copy(z_hbm.at[src_all.at[j + 1]], srows.at[nb],
                                 sems.at[nb])
                pltpu.async_copy(z_hbm.at[dst_all.at[j + 1]], drows.at[nb],
                                 semd.at[nb])
            pltpu.make_async_copy(z_hbm.at[src_all.at[j]], srows.at[b],
                                  sems.at[b]).wait()
            pltpu.make_async_copy(z_hbm.at[dst_all.at[j]], drows.at[b],
                                  semd.at[b]).wait()

            def edge(e, _):
                a = jnp.zeros((16,), F32)
                for k in range(D // 16):
                    a = a + (srows[b, e, pl.ds(k * 16, 16)]
                             * drows[b, e, pl.ds(k * 16, 16)])
                acc_v[e, pl.ds(0, 16)] = a
                return 0
            lax.fori_loop(0, CHUNK, edge, 0, unroll=4)
            pltpu.sync_copy(acc_v,
                            out_hbm.at[pl.ds((wid * K + j) * CHUNK, CHUNK)])
            return 0
        lax.fori_loop(0, K, chunk_body, 0)

    return pl.kernel(body, out_type=jax.ShapeDtypeStruct((E_PAD, 16), F32),
                     mesh=mesh, scratch_types=scratch)


def _rowsum_call():
    BLKE = 4096
    grid = E_PAD // BLKE

    def body(a_ref, out_ref):
        out_ref[...] = jnp.sum(a_ref[...], axis=1)

    return pl.pallas_call(
        body,
        grid=(grid,),
        in_specs=[pl.BlockSpec((BLKE, 16), lambda i: (i, 0))],
        out_specs=pl.BlockSpec((BLKE,), lambda i: (i,)),
        out_shape=jax.ShapeDtypeStruct((E_PAD,), F32),
    )


def kernel(x, edge_index, W1_l, b1_l, W1_r, W2_l, b2_l, W2_r):
    src = edge_index[0]
    dst = edge_index[1]
    pad = E_PAD - E
    src_p = jnp.concatenate([src, jnp.zeros((pad,), I32)]).reshape(NW * K, CHUNK)
    dst_p = jnp.concatenate([dst, jnp.full((pad,), N, I32)]).reshape(NW * K, CHUNK)
    x_p = jnp.zeros((NPAD, D), F32).at[:N].set(x)
    zrows = jnp.zeros((CHUNK, D), F32)
    orows = jnp.ones((CHUNK, D), F32)

    agg1, deg = _segsum_call(True)(x_p, src_p, dst_p, zrows, orows)
    h = _dense_call(True)(agg1, agg1, deg, deg, x_p,
                          W1_l.T, W1_r.T, b1_l[None, :])
    (agg2,) = _segsum_call(False)(h, src_p, dst_p, zrows, orows)
    z = _dense_call(False)(agg2, agg2, deg, deg, h,
                           W2_l.T, W2_r.T, b2_l[None, :])
    partial = _decode_call()(z, src_p, dst_p)
    dots = _rowsum_call()(partial)
    return dots[:E]


# speedup vs baseline: 1.0043x; 1.0043x over previous
import jax
import jax.numpy as jnp
from jax import lax
from jax.experimental import pallas as pl
from jax.experimental.pallas import tpu as pltpu
from jax.experimental.pallas import tpu_sc as plsc

N = 10000
E = 320000
D = 128

NC = 2
NS = 16
NW = NC * NS
CHUNK = 128
K = (-(-E // (NW * CHUNK)) + 7) // 8 * 8
EPT = K * CHUNK
E_PAD = NW * EPT
ROWS_PER_TILE = 640
NPAD = NS * ROWS_PER_TILE
F32 = jnp.float32
I32 = jnp.int32


def _segsum_call(with_deg):
    mesh = plsc.VectorSubcoreMesh(core_axis_name="c", subcore_axis_name="s")
    out_type = [jax.ShapeDtypeStruct((NC * NPAD, D), F32)]
    scratch = [
        pltpu.VMEM((2, CHUNK), I32),
        pltpu.VMEM((K, CHUNK), I32),
        pltpu.VMEM((2, CHUNK, D), F32),
        pltpu.VMEM_SHARED((NPAD, D), F32),
        pltpu.SemaphoreType.DMA((2,)),
        pltpu.SemaphoreType.DMA((2,)),
    ]
    if with_deg:
        out_type.append(jax.ShapeDtypeStruct((NC * NPAD, D), F32))

    def body(rows_hbm, src_hbm, dst_hbm, zrows_hbm, orows_hbm, agg_out, *rest):
        if with_deg:
            deg_out, src_v, dst_all, rows_v, agg_sh, sem_g, sem_s = rest
        else:
            src_v, dst_all, rows_v, agg_sh, sem_g, sem_s = rest
        c = lax.axis_index("c")
        s = lax.axis_index("s")
        wid = s * NC + c

        pltpu.sync_copy(dst_hbm.at[pl.ds(wid * K, K)], dst_all)

        def fill(t, _):
            pltpu.sync_copy(rows_v.at[1],
                            agg_sh.at[pl.ds(s * ROWS_PER_TILE + t * CHUNK, CHUNK)])
            return 0

        def wb(out):
            def step(t, _):
                r0 = s * ROWS_PER_TILE + t * CHUNK
                pltpu.sync_copy(agg_sh.at[pl.ds(r0, CHUNK)],
                                out.at[pl.ds(c * NPAD + r0, CHUNK)])
                return 0
            lax.fori_loop(0, ROWS_PER_TILE // CHUNK, step, 0)

        pltpu.sync_copy(zrows_hbm, rows_v.at[1])
        lax.fori_loop(0, ROWS_PER_TILE // CHUNK, fill, 0)

        if with_deg:
            pltpu.sync_copy(orows_hbm, rows_v.at[0])
            plsc.subcore_barrier()

            def deg_group(g, _):
                for i in range(8):
                    pltpu.async_copy(rows_v.at[0],
                                     agg_sh.at[dst_all.at[g * 8 + i]],
                                     sem_s.at[0], add=True)
                for i in range(8):
                    pltpu.make_async_copy(rows_v.at[0],
                                          agg_sh.at[dst_all.at[0]],
                                          sem_s.at[0]).wait()
                return 0
            lax.fori_loop(0, K // 8, deg_group, 0)
            plsc.subcore_barrier()
            wb(deg_out)
            lax.fori_loop(0, ROWS_PER_TILE // CHUNK, fill, 0)

        plsc.subcore_barrier()

        pltpu.sync_copy(src_hbm.at[pl.ds(wid * K, 1)], src_v.at[pl.ds(0, 1)])
        pltpu.async_copy(rows_hbm.at[src_v.at[0]], rows_v.at[0], sem_g.at[0])

        def chunk_body(j, _):
            b = lax.rem(j, 2)
            nb = lax.rem(j + 1, 2)

            @pl.when(j + 1 < K)
            def _():
                pltpu.sync_copy(src_hbm.at[pl.ds(wid * K + j + 1, 1)],
                                src_v.at[pl.ds(nb, 1)])

                @pl.when(j >= 1)
                def _():
                    pltpu.make_async_copy(rows_v.at[nb],
                                          agg_sh.at[dst_all.at[0]],
                                          sem_s.at[nb]).wait()
                pltpu.async_copy(rows_hbm.at[src_v.at[nb]], rows_v.at[nb],
                                 sem_g.at[nb])
            pltpu.make_async_copy(rows_hbm.at[src_v.at[b]], rows_v.at[b],
                                  sem_g.at[b]).wait()
            pltpu.async_copy(rows_v.at[b], agg_sh.at[dst_all.at[j]],
                             sem_s.at[b], add=True)
            return 0
        lax.fori_loop(0, K, chunk_body, 0)
        pltpu.make_async_copy(rows_v.at[0], agg_sh.at[dst_all.at[0]],
                              sem_s.at[0]).wait()
        pltpu.make_async_copy(rows_v.at[1], agg_sh.at[dst_all.at[0]],
                              sem_s.at[1]).wait()
        plsc.subcore_barrier()
        wb(agg_out)

    return pl.kernel(body, out_type=out_type, mesh=mesh, scratch_types=scratch)


def _dense_call(relu):
    BLK = 1024
    grid = NPAD // BLK
    half = NPAD // BLK

    def body(agg0_ref, agg1_ref, deg0_ref, deg1_ref, x_ref, wl_ref, wr_ref,
             b_ref, out_ref):
        agg = agg0_ref[...] + agg1_ref[...]
        deg = deg0_ref[:, :1] + deg1_ref[:, :1]
        mean = agg / jnp.maximum(deg, 1.0)
        h = (jnp.dot(mean, wl_ref[...], preferred_element_type=F32)
             + jnp.dot(x_ref[...], wr_ref[...], preferred_element_type=F32)
             + b_ref[...])
        out_ref[...] = jnp.maximum(h, 0.0) if relu else h

    return pl.pallas_call(
        body,
        grid=(grid,),
        in_specs=[
            pl.BlockSpec((BLK, D), lambda i: (i, 0)),
            pl.BlockSpec((BLK, D), lambda i: (half + i, 0)),
            pl.BlockSpec((BLK, D), lambda i: (i, 0)),
            pl.BlockSpec((BLK, D), lambda i: (half + i, 0)),
            pl.BlockSpec((BLK, D), lambda i: (i, 0)),
            pl.BlockSpec((D, D), lambda i: (0, 0)),
            pl.BlockSpec((D, D), lambda i: (0, 0)),
            pl.BlockSpec((1, D), lambda i: (0, 0)),
        ],
        out_specs=pl.BlockSpec((BLK, D), lambda i: (i, 0)),
        out_shape=jax.ShapeDtypeStruct((NPAD, D), F32),
    )


def _decode_call():
    mesh = plsc.VectorSubcoreMesh(core_axis_name="c", subcore_axis_name="s")
    scratch = [
        pltpu.VMEM((K, CHUNK), I32),
        pltpu.VMEM((K, CHUNK), I32),
        pltpu.VMEM((2, CHUNK, D), F32),
        pltpu.VMEM((2, CHUNK, D), F32),
        pltpu.VMEM((2, CHUNK, 16), F32),
        pltpu.SemaphoreType.DMA((2,)),
        pltpu.SemaphoreType.DMA((2,)),
        pltpu.SemaphoreType.DMA((2,)),
    ]

    def body(z_hbm, src_hbm, dst_hbm, out_hbm, src_all, dst_all, srows, drows,
             acc_v, sems, semd, semw):
        c = lax.axis_index("c")
        s = lax.axis_index("s")
        wid = s * NC + c

        pltpu.sync_copy(src_hbm.at[pl.ds(wid * K, K)], src_all)
        pltpu.sync_copy(dst_hbm.at[pl.ds(wid * K, K)], dst_all)

        pltpu.async_copy(z_hbm.at[src_all.at[0]], srows.at[0], sems.at[0])
        pltpu.async_copy(z_hbm.at[dst_all.at[0]], drows.at[0], semd.at[0])

        def chunk_body(j, _):
            b = lax.rem(j, 2)
            nb = lax.rem(j + 1, 2)

            @pl.when(j + 1 < K)
            def _():
                pltpu.async_copy(z_hbm.at[src_all.at[j + 1]], srows.at[nb],
                                 sems.at[nb])
                pltpu.async_copy(z_hbm.at[dst_all.at[j + 1]], drows.at[nb],
                                 semd.at[nb])
            pltpu.make_async_copy(z_hbm.at[src_all.at[j]], srows.at[b],
                                  sems.at[b]).wait()
            pltpu.make_async_copy(z_hbm.at[dst_all.at[j]], drows.at[b],
                                  semd.at[b]).wait()

            @pl.when(j >= 2)
            def _():
                pltpu.make_async_copy(acc_v.at[b], out_hbm.at[pl.ds(0, CHUNK)],
                                      semw.at[b]).wait()

            def edge(e, _):
                a = jnp.zeros((16,), F32)
                for k in range(D // 16):
                    a = a + (srows[b, e, pl.ds(k * 16, 16)]
                             * drows[b, e, pl.ds(k * 16, 16)])
                acc_v[b, e, pl.ds(0, 16)] = a
                return 0
            lax.fori_loop(0, CHUNK, edge, 0, unroll=8)
            pltpu.async_copy(acc_v.at[b],
                             out_hbm.at[pl.ds((wid * K + j) * CHUNK, CHUNK)],
                             semw.at[b])
            return 0
        lax.fori_loop(0, K, chunk_body, 0)
        pltpu.make_async_copy(acc_v.at[0], out_hbm.at[pl.ds(0, CHUNK)],
                              semw.at[0]).wait()
        pltpu.make_async_copy(acc_v.at[1], out_hbm.at[pl.ds(0, CHUNK)],
                              semw.at[1]).wait()

    return pl.kernel(body, out_type=jax.ShapeDtypeStruct((E_PAD, 16), F32),
                     mesh=mesh, scratch_types=scratch)


def _rowsum_call():
    BLKE = 4096
    grid = E_PAD // BLKE

    def body(a_ref, out_ref):
        out_ref[...] = jnp.sum(a_ref[...], axis=1)

    return pl.pallas_call(
        body,
        grid=(grid,),
        in_specs=[pl.BlockSpec((BLKE, 16), lambda i: (i, 0))],
        out_specs=pl.BlockSpec((BLKE,), lambda i: (i,)),
        out_shape=jax.ShapeDtypeStruct((E_PAD,), F32),
    )


def kernel(x, edge_index, W1_l, b1_l, W1_r, W2_l, b2_l, W2_r):
    src = edge_index[0]
    dst = edge_index[1]
    pad = E_PAD - E
    src_p = jnp.concatenate([src, jnp.zeros((pad,), I32)]).reshape(NW * K, CHUNK)
    dst_p = jnp.concatenate([dst, jnp.full((pad,), N, I32)]).reshape(NW * K, CHUNK)
    x_p = jnp.zeros((NPAD, D), F32).at[:N].set(x)
    zrows = jnp.zeros((CHUNK, D), F32)
    orows = jnp.ones((CHUNK, D), F32)

    agg1, deg = _segsum_call(True)(x_p, src_p, dst_p, zrows, orows)
    h = _dense_call(True)(agg1, agg1, deg, deg, x_p,
                          W1_l.T, W1_r.T, b1_l[None, :])
    (agg2,) = _segsum_call(False)(h, src_p, dst_p, zrows, orows)
    z = _dense_call(False)(agg2, agg2, deg, deg, h,
                           W2_l.T, W2_r.T, b2_l[None, :])
    partial = _decode_call()(z, src_p, dst_p)
    dots = _rowsum_call()(partial)
    return dots[:E]


# speedup vs baseline: 1.2863x; 1.2808x over previous
import jax
import jax.numpy as jnp
from jax import lax
from jax.experimental import pallas as pl
from jax.experimental.pallas import tpu as pltpu
from jax.experimental.pallas import tpu_sc as plsc

N = 10000
E = 320000
D = 128

NC = 2
NS = 16
NW = NC * NS
CHUNK = 128
K = (-(-E // (NW * CHUNK)) + 7) // 8 * 8
EPT = K * CHUNK
E_PAD = NW * EPT
ROWS_PER_TILE = 640
NPAD = NS * ROWS_PER_TILE
F32 = jnp.float32
I32 = jnp.int32


def _segsum_call(with_deg):
    mesh = plsc.VectorSubcoreMesh(core_axis_name="c", subcore_axis_name="s")
    out_type = [jax.ShapeDtypeStruct((NC * NPAD, D), F32)]
    scratch = [
        pltpu.VMEM((2, CHUNK), I32),
        pltpu.VMEM((K, CHUNK), I32),
        pltpu.VMEM((2, CHUNK, D), F32),
        pltpu.VMEM_SHARED((NPAD, D), F32),
        pltpu.SemaphoreType.DMA((2,)),
        pltpu.SemaphoreType.DMA((2,)),
    ]
    if with_deg:
        out_type.append(jax.ShapeDtypeStruct((NC * NPAD, D), F32))

    def body(rows_hbm, src_hbm, dst_hbm, zrows_hbm, orows_hbm, agg_out, *rest):
        if with_deg:
            deg_out, src_v, dst_all, rows_v, agg_sh, sem_g, sem_s = rest
        else:
            src_v, dst_all, rows_v, agg_sh, sem_g, sem_s = rest
        c = lax.axis_index("c")
        s = lax.axis_index("s")
        wid = s * NC + c

        pltpu.sync_copy(dst_hbm.at[pl.ds(wid * K, K)], dst_all)

        def fill(t, _):
            pltpu.sync_copy(rows_v.at[1],
                            agg_sh.at[pl.ds(s * ROWS_PER_TILE + t * CHUNK, CHUNK)])
            return 0

        def wb(out):
            def step(t, _):
                r0 = s * ROWS_PER_TILE + t * CHUNK
                pltpu.sync_copy(agg_sh.at[pl.ds(r0, CHUNK)],
                                out.at[pl.ds(c * NPAD + r0, CHUNK)])
                return 0
            lax.fori_loop(0, ROWS_PER_TILE // CHUNK, step, 0)

        pltpu.sync_copy(zrows_hbm, rows_v.at[1])
        lax.fori_loop(0, ROWS_PER_TILE // CHUNK, fill, 0)

        if with_deg:
            pltpu.sync_copy(orows_hbm, rows_v.at[0])
            plsc.subcore_barrier()

            def deg_group(g, _):
                for i in range(8):
                    pltpu.async_copy(rows_v.at[0],
                                     agg_sh.at[dst_all.at[g * 8 + i]],
                                     sem_s.at[0], add=True)
                for i in range(8):
                    pltpu.make_async_copy(rows_v.at[0],
                                          agg_sh.at[dst_all.at[0]],
                                          sem_s.at[0]).wait()
                return 0
            lax.fori_loop(0, K // 8, deg_group, 0)
            plsc.subcore_barrier()
            wb(deg_out)
            lax.fori_loop(0, ROWS_PER_TILE // CHUNK, fill, 0)

        plsc.subcore_barrier()

        pltpu.sync_copy(src_hbm.at[pl.ds(wid * K, 1)], src_v.at[pl.ds(0, 1)])
        pltpu.async_copy(rows_hbm.at[src_v.at[0]], rows_v.at[0], sem_g.at[0])

        def chunk_body(j, _):
            b = lax.rem(j, 2)
            nb = lax.rem(j + 1, 2)

            @pl.when(j + 1 < K)
            def _():
                pltpu.sync_copy(src_hbm.at[pl.ds(wid * K + j + 1, 1)],
                                src_v.at[pl.ds(nb, 1)])

                @pl.when(j >= 1)
                def _():
                    pltpu.make_async_copy(rows_v.at[nb],
                                          agg_sh.at[dst_all.at[0]],
                                          sem_s.at[nb]).wait()
                pltpu.async_copy(rows_hbm.at[src_v.at[nb]], rows_v.at[nb],
                                 sem_g.at[nb])
            pltpu.make_async_copy(rows_hbm.at[src_v.at[b]], rows_v.at[b],
                                  sem_g.at[b]).wait()
            pltpu.async_copy(rows_v.at[b], agg_sh.at[dst_all.at[j]],
                             sem_s.at[b], add=True)
            return 0
        lax.fori_loop(0, K, chunk_body, 0)
        pltpu.make_async_copy(rows_v.at[0], agg_sh.at[dst_all.at[0]],
                              sem_s.at[0]).wait()
        pltpu.make_async_copy(rows_v.at[1], agg_sh.at[dst_all.at[0]],
                              sem_s.at[1]).wait()
        plsc.subcore_barrier()
        wb(agg_out)

    return pl.kernel(body, out_type=out_type, mesh=mesh, scratch_types=scratch)


def _dense_call(relu):
    BLK = 1024
    grid = NPAD // BLK
    half = NPAD // BLK

    def body(agg0_ref, agg1_ref, deg0_ref, deg1_ref, x_ref, wl_ref, wr_ref,
             b_ref, out_ref):
        agg = agg0_ref[...] + agg1_ref[...]
        deg = deg0_ref[:, :1] + deg1_ref[:, :1]
        mean = agg / jnp.maximum(deg, 1.0)
        h = (jnp.dot(mean, wl_ref[...], preferred_element_type=F32)
             + jnp.dot(x_ref[...], wr_ref[...], preferred_element_type=F32)
             + b_ref[...])
        out_ref[...] = jnp.maximum(h, 0.0) if relu else h

    return pl.pallas_call(
        body,
        grid=(grid,),
        in_specs=[
            pl.BlockSpec((BLK, D), lambda i: (i, 0)),
            pl.BlockSpec((BLK, D), lambda i: (half + i, 0)),
            pl.BlockSpec((BLK, D), lambda i: (i, 0)),
            pl.BlockSpec((BLK, D), lambda i: (half + i, 0)),
            pl.BlockSpec((BLK, D), lambda i: (i, 0)),
            pl.BlockSpec((D, D), lambda i: (0, 0)),
            pl.BlockSpec((D, D), lambda i: (0, 0)),
            pl.BlockSpec((1, D), lambda i: (0, 0)),
        ],
        out_specs=pl.BlockSpec((BLK, D), lambda i: (i, 0)),
        out_shape=jax.ShapeDtypeStruct((NPAD, D), F32),
    )


def _decode_call():
    mesh = plsc.VectorSubcoreMesh(core_axis_name="c", subcore_axis_name="s")
    scratch = [
        pltpu.VMEM((K, CHUNK), I32),
        pltpu.VMEM((K, CHUNK), I32),
        pltpu.VMEM((2, CHUNK, D), F32),
        pltpu.VMEM((2, CHUNK, D), F32),
        pltpu.VMEM((2, CHUNK, 16), F32),
        pltpu.SemaphoreType.DMA((2,)),
        pltpu.SemaphoreType.DMA((2,)),
        pltpu.SemaphoreType.DMA((2,)),
    ]

    def body(z_hbm, src_hbm, dst_hbm, out_hbm, src_all, dst_all, srows, drows,
             acc_v, sems, semd, semw):
        c = lax.axis_index("c")
        s = lax.axis_index("s")
        wid = s * NC + c

        pltpu.sync_copy(src_hbm.at[pl.ds(wid * K, K)], src_all)
        pltpu.sync_copy(dst_hbm.at[pl.ds(wid * K, K)], dst_all)

        pltpu.async_copy(z_hbm.at[src_all.at[0]], srows.at[0], sems.at[0])
        pltpu.async_copy(z_hbm.at[dst_all.at[0]], drows.at[0], semd.at[0])

        def chunk_body(j, _):
            b = lax.rem(j, 2)
            nb = lax.rem(j + 1, 2)

            @pl.when(j + 1 < K)
            def _():
                pltpu.async_copy(z_hbm.at[src_all.at[j + 1]], srows.at[nb],
                                 sems.at[nb])
                pltpu.async_copy(z_hbm.at[dst_all.at[j + 1]], drows.at[nb],
                                 semd.at[nb])
            pltpu.make_async_copy(z_hbm.at[src_all.at[j]], srows.at[b],
                                  sems.at[b]).wait()
            pltpu.make_async_copy(z_hbm.at[dst_all.at[j]], drows.at[b],
                                  semd.at[b]).wait()

            @pl.when(j >= 2)
            def _():
                pltpu.make_async_copy(acc_v.at[b], out_hbm.at[pl.ds(0, CHUNK)],
                                      semw.at[b]).wait()

            def edge(e, _):
                a = jnp.zeros((16,), F32)
                for k in range(D // 16):
                    a = a + (srows[b, e, pl.ds(k * 16, 16)]
                             * drows[b, e, pl.ds(k * 16, 16)])
                acc_v[b, e, pl.ds(0, 16)] = a
                return 0
            lax.fori_loop(0, CHUNK, edge, 0, unroll=8)
            pltpu.async_copy(acc_v.at[b],
                             out_hbm.at[pl.ds((wid * K + j) * CHUNK, CHUNK)],
                             semw.at[b])
            return 0
        lax.fori_loop(0, K, chunk_body, 0)
        pltpu.make_async_copy(acc_v.at[0], out_hbm.at[pl.ds(0, CHUNK)],
                              semw.at[0]).wait()
        pltpu.make_async_copy(acc_v.at[1], out_hbm.at[pl.ds(0, CHUNK)],
                              semw.at[1]).wait()

    return pl.kernel(body, out_type=jax.ShapeDtypeStruct((E_PAD, 16), F32),
                     mesh=mesh, scratch_types=scratch)


def _rowsum_call():
    BLKE = 4096
    grid = E_PAD // BLKE

    def body(a_ref, out_ref):
        out_ref[...] = jnp.sum(a_ref[...], axis=1)

    return pl.pallas_call(
        body,
        grid=(grid,),
        in_specs=[pl.BlockSpec((BLKE, 16), lambda i: (i, 0))],
        out_specs=pl.BlockSpec((BLKE,), lambda i: (i,)),
        out_shape=jax.ShapeDtypeStruct((E_PAD,), F32),
    )


def kernel(x, edge_index, W1_l, b1_l, W1_r, W2_l, b2_l, W2_r):
    src = edge_index[0]
    dst = edge_index[1]
    ept_real = E // NW
    padn = EPT - ept_real
    pad_dst = jnp.broadcast_to(jnp.arange(N, N + padn, dtype=I32), (NW, padn))
    pad_src = jnp.zeros((NW, padn), I32)
    src_p = jnp.concatenate([src.reshape(NW, ept_real), pad_src],
                            axis=1).reshape(NW * K, CHUNK)
    dst_p = jnp.concatenate([dst.reshape(NW, ept_real), pad_dst],
                            axis=1).reshape(NW * K, CHUNK)
    x_p = jnp.zeros((NPAD, D), F32).at[:N].set(x)
    zrows = jnp.zeros((CHUNK, D), F32)
    orows = jnp.ones((CHUNK, D), F32)

    agg1, deg = _segsum_call(True)(x_p, src_p, dst_p, zrows, orows)
    h = _dense_call(True)(agg1, agg1, deg, deg, x_p,
                          W1_l.T, W1_r.T, b1_l[None, :])
    (agg2,) = _segsum_call(False)(h, src_p, dst_p, zrows, orows)
    z = _dense_call(False)(agg2, agg2, deg, deg, h,
                           W2_l.T, W2_r.T, b2_l[None, :])
    partial = _decode_call()(z, src_p, dst_p)
    dots = _rowsum_call()(partial)
    return dots.reshape(NW, EPT)[:, :ept_real].reshape(E)


# speedup vs baseline: 1.2880x; 1.0013x over previous
import jax
import jax.numpy as jnp
from jax import lax
from jax.experimental import pallas as pl
from jax.experimental.pallas import tpu as pltpu
from jax.experimental.pallas import tpu_sc as plsc

N = 10000
E = 320000
D = 128

NC = 2
NS = 16
NW = NC * NS
CHUNK = 128
K = (-(-E // (NW * CHUNK)) + 7) // 8 * 8
EPT = K * CHUNK
E_PAD = NW * EPT
ROWS_PER_TILE = 640
NPAD = NS * ROWS_PER_TILE
F32 = jnp.float32
I32 = jnp.int32


def _segsum_call(with_deg):
    mesh = plsc.VectorSubcoreMesh(core_axis_name="c", subcore_axis_name="s")
    out_type = [jax.ShapeDtypeStruct((NC * NPAD, D), F32)]
    scratch = [
        pltpu.VMEM((2, CHUNK), I32),
        pltpu.VMEM((K, CHUNK), I32),
        pltpu.VMEM((2, CHUNK, D), F32),
        pltpu.VMEM_SHARED((NPAD, D), F32),
        pltpu.SemaphoreType.DMA((2,)),
        pltpu.SemaphoreType.DMA((2,)),
    ]
    if with_deg:
        out_type.append(jax.ShapeDtypeStruct((NC * NPAD, D), F32))

    def body(rows_hbm, src_hbm, dst_hbm, zrows_hbm, orows_hbm, agg_out, *rest):
        if with_deg:
            deg_out, src_v, dst_all, rows_v, agg_sh, sem_g, sem_s = rest
        else:
            src_v, dst_all, rows_v, agg_sh, sem_g, sem_s = rest
        c = lax.axis_index("c")
        s = lax.axis_index("s")
        wid = s * NC + c

        pltpu.sync_copy(dst_hbm.at[pl.ds(wid * K, K)], dst_all)

        def fill(t, _):
            pltpu.sync_copy(rows_v.at[1],
                            agg_sh.at[pl.ds(s * ROWS_PER_TILE + t * CHUNK, CHUNK)])
            return 0

        def wb(out):
            def step(t, _):
                r0 = s * ROWS_PER_TILE + t * CHUNK
                pltpu.sync_copy(agg_sh.at[pl.ds(r0, CHUNK)],
                                out.at[pl.ds(c * NPAD + r0, CHUNK)])
                return 0
            lax.fori_loop(0, ROWS_PER_TILE // CHUNK, step, 0)

        pltpu.sync_copy(zrows_hbm, rows_v.at[1])
        lax.fori_loop(0, ROWS_PER_TILE // CHUNK, fill, 0)

        if with_deg:
            pltpu.sync_copy(orows_hbm, rows_v.at[0])
            plsc.subcore_barrier()

            def deg_group(g, _):
                for i in range(8):
                    pltpu.async_copy(rows_v.at[0],
                                     agg_sh.at[dst_all.at[g * 8 + i]],
                                     sem_s.at[0], add=True)
                for i in range(8):
                    pltpu.make_async_copy(rows_v.at[0],
                                          agg_sh.at[dst_all.at[0]],
                                          sem_s.at[0]).wait()
                return 0
            lax.fori_loop(0, K // 8, deg_group, 0)
            plsc.subcore_barrier()
            wb(deg_out)
            lax.fori_loop(0, ROWS_PER_TILE // CHUNK, fill, 0)

        plsc.subcore_barrier()

        pltpu.sync_copy(src_hbm.at[pl.ds(wid * K, 1)], src_v.at[pl.ds(0, 1)])
        pltpu.async_copy(rows_hbm.at[src_v.at[0]], rows_v.at[0], sem_g.at[0])

        def chunk_body(j, _):
            b = lax.rem(j, 2)
            nb = lax.rem(j + 1, 2)

            @pl.when(j + 1 < K)
            def _():
                pltpu.sync_copy(src_hbm.at[pl.ds(wid * K + j + 1, 1)],
                                src_v.at[pl.ds(nb, 1)])

                @pl.when(j >= 1)
                def _():
                    pltpu.make_async_copy(rows_v.at[nb],
                                          agg_sh.at[dst_all.at[0]],
                                          sem_s.at[nb]).wait()
                pltpu.async_copy(rows_hbm.at[src_v.at[nb]], rows_v.at[nb],
                                 sem_g.at[nb])
            pltpu.make_async_copy(rows_hbm.at[src_v.at[b]], rows_v.at[b],
                                  sem_g.at[b]).wait()
            pltpu.async_copy(rows_v.at[b], agg_sh.at[dst_all.at[j]],
                             sem_s.at[b], add=True)
            return 0
        lax.fori_loop(0, K, chunk_body, 0)
        pltpu.make_async_copy(rows_v.at[0], agg_sh.at[dst_all.at[0]],
                              sem_s.at[0]).wait()
        pltpu.make_async_copy(rows_v.at[1], agg_sh.at[dst_all.at[0]],
                              sem_s.at[1]).wait()
        plsc.subcore_barrier()
        wb(agg_out)

    return pl.kernel(body, out_type=out_type, mesh=mesh, scratch_types=scratch)


def _dense_call(relu):
    BLK = 1024
    grid = NPAD // BLK
    half = NPAD // BLK

    def body(agg0_ref, agg1_ref, deg0_ref, deg1_ref, x_ref, wl_ref, wr_ref,
             b_ref, out_ref):
        agg = agg0_ref[...] + agg1_ref[...]
        deg = deg0_ref[:, :1] + deg1_ref[:, :1]
        mean = agg / jnp.maximum(deg, 1.0)
        h = (jnp.dot(mean, wl_ref[...], preferred_element_type=F32)
             + jnp.dot(x_ref[...], wr_ref[...], preferred_element_type=F32)
             + b_ref[...])
        out_ref[...] = jnp.maximum(h, 0.0) if relu else h

    return pl.pallas_call(
        body,
        grid=(grid,),
        in_specs=[
            pl.BlockSpec((BLK, D), lambda i: (i, 0)),
            pl.BlockSpec((BLK, D), lambda i: (half + i, 0)),
            pl.BlockSpec((BLK, D), lambda i: (i, 0)),
            pl.BlockSpec((BLK, D), lambda i: (half + i, 0)),
            pl.BlockSpec((BLK, D), lambda i: (i, 0)),
            pl.BlockSpec((D, D), lambda i: (0, 0)),
            pl.BlockSpec((D, D), lambda i: (0, 0)),
            pl.BlockSpec((1, D), lambda i: (0, 0)),
        ],
        out_specs=pl.BlockSpec((BLK, D), lambda i: (i, 0)),
        out_shape=jax.ShapeDtypeStruct((NPAD, D), F32),
    )


def _decode_call():
    mesh = plsc.VectorSubcoreMesh(core_axis_name="c", subcore_axis_name="s")
    scratch = [
        pltpu.VMEM((K, CHUNK), I32),
        pltpu.VMEM((K, CHUNK), I32),
        pltpu.VMEM((2, CHUNK, D), F32),
        pltpu.VMEM((2, CHUNK, D), F32),
        pltpu.VMEM((2, CHUNK, 16), F32),
        pltpu.SemaphoreType.DMA((2,)),
        pltpu.SemaphoreType.DMA((2,)),
        pltpu.SemaphoreType.DMA((2,)),
    ]

    def body(z_hbm, src_hbm, dst_hbm, out_hbm, src_all, dst_all, srows, drows,
             acc_v, sems, semd, semw):
        c = lax.axis_index("c")
        s = lax.axis_index("s")
        wid = s * NC + c

        pltpu.sync_copy(src_hbm.at[pl.ds(wid * K, K)], src_all)
        pltpu.sync_copy(dst_hbm.at[pl.ds(wid * K, K)], dst_all)

        pltpu.async_copy(z_hbm.at[src_all.at[0]], srows.at[0], sems.at[0])
        pltpu.async_copy(z_hbm.at[dst_all.at[0]], drows.at[0], semd.at[0])

        def chunk_body(j, _):
            b = lax.rem(j, 2)
            nb = lax.rem(j + 1, 2)

            @pl.when(j + 1 < K)
            def _():
                pltpu.async_copy(z_hbm.at[src_all.at[j + 1]], srows.at[nb],
                                 sems.at[nb])
                pltpu.async_copy(z_hbm.at[dst_all.at[j + 1]], drows.at[nb],
                                 semd.at[nb])
            pltpu.make_async_copy(z_hbm.at[src_all.at[j]], srows.at[b],
                                  sems.at[b]).wait()
            pltpu.make_async_copy(z_hbm.at[dst_all.at[j]], drows.at[b],
                                  semd.at[b]).wait()

            @pl.when(j >= 2)
            def _():
                pltpu.make_async_copy(acc_v.at[b], out_hbm.at[pl.ds(0, CHUNK)],
                                      semw.at[b]).wait()

            @plsc.parallel_loop(0, CHUNK, unroll=8)
            def _(e):
                a = jnp.zeros((16,), F32)
                for k in range(D // 16):
                    a = a + (srows[b, e, pl.ds(k * 16, 16)]
                             * drows[b, e, pl.ds(k * 16, 16)])
                acc_v[b, e, pl.ds(0, 16)] = a
            pltpu.async_copy(acc_v.at[b],
                             out_hbm.at[pl.ds((wid * K + j) * CHUNK, CHUNK)],
                             semw.at[b])
            return 0
        lax.fori_loop(0, K, chunk_body, 0)
        pltpu.make_async_copy(acc_v.at[0], out_hbm.at[pl.ds(0, CHUNK)],
                              semw.at[0]).wait()
        pltpu.make_async_copy(acc_v.at[1], out_hbm.at[pl.ds(0, CHUNK)],
                              semw.at[1]).wait()

    return pl.kernel(body, out_type=jax.ShapeDtypeStruct((E_PAD, 16), F32),
                     mesh=mesh, scratch_types=scratch)


def _rowsum_call():
    BLKE = 4096
    grid = E_PAD // BLKE

    def body(a_ref, out_ref):
        out_ref[...] = jnp.sum(a_ref[...], axis=1)

    return pl.pallas_call(
        body,
        grid=(grid,),
        in_specs=[pl.BlockSpec((BLKE, 16), lambda i: (i, 0))],
        out_specs=pl.BlockSpec((BLKE,), lambda i: (i,)),
        out_shape=jax.ShapeDtypeStruct((E_PAD,), F32),
    )


def kernel(x, edge_index, W1_l, b1_l, W1_r, W2_l, b2_l, W2_r):
    src = edge_index[0]
    dst = edge_index[1]
    ept_real = E // NW
    padn = EPT - ept_real
    pad_dst = jnp.broadcast_to(jnp.arange(N, N + padn, dtype=I32), (NW, padn))
    pad_src = jnp.zeros((NW, padn), I32)
    src_p = jnp.concatenate([src.reshape(NW, ept_real), pad_src],
                            axis=1).reshape(NW * K, CHUNK)
    dst_p = jnp.concatenate([dst.reshape(NW, ept_real), pad_dst],
                            axis=1).reshape(NW * K, CHUNK)
    x_p = jnp.zeros((NPAD, D), F32).at[:N].set(x)
    zrows = jnp.zeros((CHUNK, D), F32)
    orows = jnp.ones((CHUNK, D), F32)

    agg1, deg = _segsum_call(True)(x_p, src_p, dst_p, zrows, orows)
    h = _dense_call(True)(agg1, agg1, deg, deg, x_p,
                          W1_l.T, W1_r.T, b1_l[None, :])
    (agg2,) = _segsum_call(False)(h, src_p, dst_p, zrows, orows)
    z = _dense_call(False)(agg2, agg2, deg, deg, h,
                           W2_l.T, W2_r.T, b2_l[None, :])
    partial = _decode_call()(z, src_p, dst_p)
    dots = _rowsum_call()(partial)
    return dots.reshape(NW, EPT)[:, :ept_real].reshape(E)
